# baseline (speedup 1.0000x reference)
import functools

import jax
import jax.numpy as jnp
from jax.experimental import pallas as pl
from jax.experimental.pallas import tpu as pltpu


def _fprop_kernel(xyz1a_ref, xyz2a_ref, a2_ref, b2_ref, f1_ref, f2_ref,
                  wi_ref, wf_ref, y_ref, stats_ref, g2_ref):
    b = pl.program_id(0)
    t = pl.program_id(1)

    @pl.when(jnp.logical_and(b == 0, t == 0))
    def _init_stats():
        stats_ref[...] = jnp.zeros_like(stats_ref)

    @pl.when(t == 0)
    def _compute_g2():
        g2_ref[...] = jnp.dot(wi_ref[...], f2_ref[0],
                              preferred_element_type=jnp.float32
                              ).astype(jnp.bfloat16)

    cross = jnp.dot(xyz2a_ref[0], xyz1a_ref[0],
                    preferred_element_type=jnp.float32)
    m = cross + (b2_ref[0] + a2_ref[0])

    v1 = jnp.min(m, axis=0, keepdims=True)
    m2 = jnp.where(m > v1, m, jnp.float32(3.0e38))
    v2 = jnp.min(m2, axis=0, keepdims=True)
    m3 = jnp.where(m2 > v2, m2, jnp.float32(3.0e38))
    v3 = jnp.min(m3, axis=0, keepdims=True)

    w0 = 1.0 / (jnp.sqrt(jnp.maximum(v1, 1e-12)) + 1e-8)
    w1 = 1.0 / (jnp.sqrt(jnp.maximum(v2, 1e-12)) + 1e-8)
    w2 = 1.0 / (jnp.sqrt(jnp.maximum(v3, 1e-12)) + 1e-8)
    ws = 1.0 / (w0 + w1 + w2)
    w0 = w0 * ws
    w1 = w1 * ws
    w2 = w2 * ws

    st = jnp.where(m == v1, w0, 0.0)
    st = jnp.where(m == v2, w1, st)
    st = jnp.where(m == v3, w2, st)

    y = jnp.dot(g2_ref[...], st.astype(jnp.bfloat16),
                preferred_element_type=jnp.float32)
    y = y + jnp.dot(wf_ref[...], f1_ref[0], preferred_element_type=jnp.float32)
    y_ref[0] = y
    stats_ref[:, 0:1] += jnp.sum(y, axis=1, keepdims=True)
    stats_ref[:, 1:2] += jnp.sum(y * y, axis=1, keepdims=True)


def _bn_kernel(y_ref, stats_ref, gamma_ref, beta_ref, o_ref, *, count):
    s1 = stats_ref[:, 0:1]
    s2 = stats_ref[:, 1:2]
    mean = s1 * (1.0 / count)
    var = s2 * (1.0 / count) - mean * mean
    a = gamma_ref[...] * jax.lax.rsqrt(var + 1e-5)
    c = beta_ref[...] - a * mean
    o_ref[0] = jnp.maximum(y_ref[0] * a + c, 0.0)


def kernel(xyz1, xyz2, feats1, feats2, W, gamma, beta):
    B, N1, _ = xyz1.shape
    N2 = xyz2.shape[1]
    C1 = feats1.shape[1]
    C2 = feats2.shape[1]
    OUT = W.shape[0]
    TN = 512 if N1 % 512 == 0 else N1
    NT = N1 // TN

    xyz1a = jnp.concatenate(
        [xyz1, jnp.zeros((B, N1, 5), xyz1.dtype)], axis=-1)
    xyz1a = jnp.transpose(xyz1a, (0, 2, 1))
    a2 = jnp.sum(xyz1 * xyz1, axis=-1)[:, None, :]
    b2 = jnp.sum(xyz2 * xyz2, axis=-1, keepdims=True)
    xyz2a = jnp.concatenate(
        [-2.0 * xyz2, jnp.zeros((B, N2, 5), xyz2.dtype)], axis=-1)
    Wi = W[:, :C2]
    Wf = W[:, C2:]

    y, stats = pl.pallas_call(
        _fprop_kernel,
        grid=(B, NT),
        in_specs=[
            pl.BlockSpec((1, 8, TN), lambda b, t: (b, 0, t)),
            pl.BlockSpec((1, N2, 8), lambda b, t: (b, 0, 0)),
            pl.BlockSpec((1, 1, TN), lambda b, t: (b, 0, t)),
            pl.BlockSpec((1, N2, 1), lambda b, t: (b, 0, 0)),
            pl.BlockSpec((1, C1, TN), lambda b, t: (b, 0, t)),
            pl.BlockSpec((1, C2, N2), lambda b, t: (b, 0, 0)),
            pl.BlockSpec((OUT, C2), lambda b, t: (0, 0)),
            pl.BlockSpec((OUT, C1), lambda b, t: (0, 0)),
        ],
        out_specs=[
            pl.BlockSpec((1, OUT, TN), lambda b, t: (b, 0, t)),
            pl.BlockSpec((OUT, 2), lambda b, t: (0, 0)),
        ],
        out_shape=[
            jax.ShapeDtypeStruct((B, OUT, N1), jnp.float32),
            jax.ShapeDtypeStruct((OUT, 2), jnp.float32),
        ],
        scratch_shapes=[pltpu.VMEM((OUT, N2), jnp.bfloat16)],
    )(xyz1a, xyz2a, a2, b2, feats1, feats2, Wi, Wf)

    out = pl.pallas_call(
        functools.partial(_bn_kernel, count=float(B * N1)),
        grid=(B, NT),
        in_specs=[
            pl.BlockSpec((1, OUT, TN), lambda b, t: (b, 0, t)),
            pl.BlockSpec((OUT, 2), lambda b, t: (0, 0)),
            pl.BlockSpec((OUT, 1), lambda b, t: (0, 0)),
            pl.BlockSpec((OUT, 1), lambda b, t: (0, 0)),
        ],
        out_specs=pl.BlockSpec((1, OUT, TN), lambda b, t: (b, 0, t)),
        out_shape=jax.ShapeDtypeStruct((B, OUT, N1), jnp.float32),
    )(y, stats, gamma.reshape(OUT, 1), beta.reshape(OUT, 1))
    return out


# speedup vs baseline: 1.0043x; 1.0043x over previous
import functools

import jax
import jax.numpy as jnp
from jax.experimental import pallas as pl
from jax.experimental.pallas import tpu as pltpu


def _tree_min(x):
    n = x.shape[1]
    while x.shape[0] > 8:
        g = 8 if x.shape[0] % 64 == 0 else 2
        x = jnp.min(x.reshape(g, x.shape[0] // g, n), axis=0)
    return jnp.min(x, axis=0, keepdims=True)


def _fprop_kernel(xyz1a_ref, xyz2a_ref, a2_ref, b2_ref, f1_ref, f2_ref,
                  wi_ref, wf_ref, y_ref, stats_ref, g2_ref):
    b = pl.program_id(0)
    t = pl.program_id(1)

    @pl.when(jnp.logical_and(b == 0, t == 0))
    def _init_stats():
        stats_ref[...] = jnp.zeros_like(stats_ref)

    @pl.when(t == 0)
    def _compute_g2():
        g2_ref[...] = jnp.dot(wi_ref[...], f2_ref[0],
                              preferred_element_type=jnp.float32)

    cross = jnp.dot(xyz2a_ref[0], xyz1a_ref[0],
                    preferred_element_type=jnp.float32)
    m = cross + b2_ref[0]

    v1 = _tree_min(m)
    m2 = jnp.where(m > v1, m, jnp.float32(3.0e38))
    v2 = _tree_min(m2)
    m3 = jnp.where(m2 > v2, m2, jnp.float32(3.0e38))
    v3 = _tree_min(m3)

    a2 = a2_ref[0]
    w0 = 1.0 / (jnp.sqrt(jnp.maximum(v1 + a2, 1e-12)) + 1e-8)
    w1 = 1.0 / (jnp.sqrt(jnp.maximum(v2 + a2, 1e-12)) + 1e-8)
    w2 = 1.0 / (jnp.sqrt(jnp.maximum(v3 + a2, 1e-12)) + 1e-8)
    ws = 1.0 / (w0 + w1 + w2)
    w0 = w0 * ws
    w1 = w1 * ws
    w2 = w2 * ws

    st = jnp.where(m == v1, w0, 0.0)
    st = jnp.where(m == v2, w1, st)
    st = jnp.where(m == v3, w2, st)

    y = jnp.dot(g2_ref[...], st, preferred_element_type=jnp.float32)
    y = y + jnp.dot(wf_ref[...], f1_ref[0], preferred_element_type=jnp.float32)
    y_ref[0] = y
    stats_ref[:, 0:1] += jnp.sum(y, axis=1, keepdims=True)
    stats_ref[:, 1:2] += jnp.sum(y * y, axis=1, keepdims=True)


def _bn_kernel(y_ref, stats_ref, gamma_ref, beta_ref, o_ref, *, count):
    s1 = stats_ref[:, 0:1]
    s2 = stats_ref[:, 1:2]
    mean = s1 * (1.0 / count)
    var = s2 * (1.0 / count) - mean * mean
    a = gamma_ref[...] * jax.lax.rsqrt(var + 1e-5)
    c = beta_ref[...] - a * mean
    o_ref[0] = jnp.maximum(y_ref[0] * a + c, 0.0)


def kernel(xyz1, xyz2, feats1, feats2, W, gamma, beta):
    B, N1, _ = xyz1.shape
    N2 = xyz2.shape[1]
    C1 = feats1.shape[1]
    C2 = feats2.shape[1]
    OUT = W.shape[0]
    TN = 512 if N1 % 512 == 0 else N1
    NT = N1 // TN

    xyz1a = jnp.concatenate(
        [xyz1, jnp.zeros((B, N1, 5), xyz1.dtype)], axis=-1)
    xyz1a = jnp.transpose(xyz1a, (0, 2, 1))
    a2 = jnp.sum(xyz1 * xyz1, axis=-1)[:, None, :]
    b2 = jnp.sum(xyz2 * xyz2, axis=-1, keepdims=True)
    xyz2a = jnp.concatenate(
        [-2.0 * xyz2, jnp.zeros((B, N2, 5), xyz2.dtype)], axis=-1)
    Wi = W[:, :C2]
    Wf = W[:, C2:]

    y, stats = pl.pallas_call(
        _fprop_kernel,
        grid=(B, NT),
        in_specs=[
            pl.BlockSpec((1, 8, TN), lambda b, t: (b, 0, t)),
            pl.BlockSpec((1, N2, 8), lambda b, t: (b, 0, 0)),
            pl.BlockSpec((1, 1, TN), lambda b, t: (b, 0, t)),
            pl.BlockSpec((1, N2, 1), lambda b, t: (b, 0, 0)),
            pl.BlockSpec((1, C1, TN), lambda b, t: (b, 0, t)),
            pl.BlockSpec((1, C2, N2), lambda b, t: (b, 0, 0)),
            pl.BlockSpec((OUT, C2), lambda b, t: (0, 0)),
            pl.BlockSpec((OUT, C1), lambda b, t: (0, 0)),
        ],
        out_specs=[
            pl.BlockSpec((1, OUT, TN), lambda b, t: (b, 0, t)),
            pl.BlockSpec((OUT, 2), lambda b, t: (0, 0)),
        ],
        out_shape=[
            jax.ShapeDtypeStruct((B, OUT, N1), jnp.float32),
            jax.ShapeDtypeStruct((OUT, 2), jnp.float32),
        ],
        scratch_shapes=[pltpu.VMEM((OUT, N2), jnp.float32)],
    )(xyz1a, xyz2a, a2, b2, feats1, feats2, Wi, Wf)

    out = pl.pallas_call(
        functools.partial(_bn_kernel, count=float(B * N1)),
        grid=(B, NT),
        in_specs=[
            pl.BlockSpec((1, OUT, TN), lambda b, t: (b, 0, t)),
            pl.BlockSpec((OUT, 2), lambda b, t: (0, 0)),
            pl.BlockSpec((OUT, 1), lambda b, t: (0, 0)),
            pl.BlockSpec((OUT, 1), lambda b, t: (0, 0)),
        ],
        out_specs=pl.BlockSpec((1, OUT, TN), lambda b, t: (b, 0, t)),
        out_shape=jax.ShapeDtypeStruct((B, OUT, N1), jnp.float32),
    )(y, stats, gamma.reshape(OUT, 1), beta.reshape(OUT, 1))
    return out


# speedup vs baseline: 1.0191x; 1.0147x over previous
import functools

import jax
import jax.numpy as jnp
from jax.experimental import pallas as pl
from jax.experimental.pallas import tpu as pltpu


def _tree_min(x):
    n = x.shape[1]
    while x.shape[0] > 8:
        g = 8 if x.shape[0] % 64 == 0 else 2
        x = jnp.min(x.reshape(g, x.shape[0] // g, n), axis=0)
    return jnp.min(x, axis=0, keepdims=True)


def _fprop_kernel(xyz1a_ref, xyz2a_ref, a2_ref, b2_ref, f1_ref, f2_ref,
                  wi_ref, wf_ref, y_ref, stats_ref, g2_ref):
    b = pl.program_id(0)
    t = pl.program_id(1)

    @pl.when(jnp.logical_and(b == 0, t == 0))
    def _init_stats():
        stats_ref[...] = jnp.zeros_like(stats_ref)

    @pl.when(t == 0)
    def _compute_g2():
        g2_ref[...] = jnp.dot(wi_ref[...], f2_ref[0],
                              preferred_element_type=jnp.float32)

    cross = jnp.dot(xyz2a_ref[0], xyz1a_ref[0],
                    preferred_element_type=jnp.float32)
    m = cross + b2_ref[0]

    v1 = _tree_min(m)
    m2 = jnp.where(m > v1, m, jnp.float32(3.0e38))
    v2 = _tree_min(m2)
    m3 = jnp.where(m2 > v2, m2, jnp.float32(3.0e38))
    v3 = _tree_min(m3)

    a2 = a2_ref[0]
    w0 = 1.0 / (jnp.sqrt(jnp.maximum(v1 + a2, 1e-12)) + 1e-8)
    w1 = 1.0 / (jnp.sqrt(jnp.maximum(v2 + a2, 1e-12)) + 1e-8)
    w2 = 1.0 / (jnp.sqrt(jnp.maximum(v3 + a2, 1e-12)) + 1e-8)
    ws = 1.0 / (w0 + w1 + w2)
    w0 = w0 * ws
    w1 = w1 * ws
    w2 = w2 * ws

    st = jnp.where(m == v1, w0, 0.0)
    st = jnp.where(m == v2, w1, st)
    st = jnp.where(m == v3, w2, st)

    y = jnp.dot(g2_ref[...], st, preferred_element_type=jnp.float32)
    y = y + jnp.dot(wf_ref[...], f1_ref[0], preferred_element_type=jnp.float32)
    y_ref[0] = y.astype(jnp.bfloat16)
    stats_ref[:, 0:1] += jnp.sum(y, axis=1, keepdims=True)
    stats_ref[:, 1:2] += jnp.sum(y * y, axis=1, keepdims=True)


def _bn_kernel(y_ref, stats_ref, gamma_ref, beta_ref, o_ref, *, count):
    s1 = stats_ref[:, 0:1]
    s2 = stats_ref[:, 1:2]
    mean = s1 * (1.0 / count)
    var = s2 * (1.0 / count) - mean * mean
    a = gamma_ref[...] * jax.lax.rsqrt(var + 1e-5)
    c = beta_ref[...] - a * mean
    o_ref[0] = jnp.maximum(y_ref[0].astype(jnp.float32) * a + c, 0.0)


def kernel(xyz1, xyz2, feats1, feats2, W, gamma, beta):
    B, N1, _ = xyz1.shape
    N2 = xyz2.shape[1]
    C1 = feats1.shape[1]
    C2 = feats2.shape[1]
    OUT = W.shape[0]
    TN = 512 if N1 % 512 == 0 else N1
    NT = N1 // TN

    xyz1a = jnp.concatenate(
        [xyz1, jnp.zeros((B, N1, 5), xyz1.dtype)], axis=-1)
    xyz1a = jnp.transpose(xyz1a, (0, 2, 1))
    a2 = jnp.sum(xyz1 * xyz1, axis=-1)[:, None, :]
    b2 = jnp.sum(xyz2 * xyz2, axis=-1, keepdims=True)
    xyz2a = jnp.concatenate(
        [-2.0 * xyz2, jnp.zeros((B, N2, 5), xyz2.dtype)], axis=-1)
    Wi = W[:, :C2]
    Wf = W[:, C2:]

    y, stats = pl.pallas_call(
        _fprop_kernel,
        grid=(B, NT),
        in_specs=[
            pl.BlockSpec((1, 8, TN), lambda b, t: (b, 0, t)),
            pl.BlockSpec((1, N2, 8), lambda b, t: (b, 0, 0)),
            pl.BlockSpec((1, 1, TN), lambda b, t: (b, 0, t)),
            pl.BlockSpec((1, N2, 1), lambda b, t: (b, 0, 0)),
            pl.BlockSpec((1, C1, TN), lambda b, t: (b, 0, t)),
            pl.BlockSpec((1, C2, N2), lambda b, t: (b, 0, 0)),
            pl.BlockSpec((OUT, C2), lambda b, t: (0, 0)),
            pl.BlockSpec((OUT, C1), lambda b, t: (0, 0)),
        ],
        out_specs=[
            pl.BlockSpec((1, OUT, TN), lambda b, t: (b, 0, t)),
            pl.BlockSpec((OUT, 2), lambda b, t: (0, 0)),
        ],
        out_shape=[
            jax.ShapeDtypeStruct((B, OUT, N1), jnp.bfloat16),
            jax.ShapeDtypeStruct((OUT, 2), jnp.float32),
        ],
        scratch_shapes=[pltpu.VMEM((OUT, N2), jnp.float32)],
    )(xyz1a, xyz2a, a2, b2, feats1, feats2, Wi, Wf)

    out = pl.pallas_call(
        functools.partial(_bn_kernel, count=float(B * N1)),
        grid=(B, NT),
        in_specs=[
            pl.BlockSpec((1, OUT, TN), lambda b, t: (b, 0, t)),
            pl.BlockSpec((OUT, 2), lambda b, t: (0, 0)),
            pl.BlockSpec((OUT, 1), lambda b, t: (0, 0)),
            pl.BlockSpec((OUT, 1), lambda b, t: (0, 0)),
        ],
        out_specs=pl.BlockSpec((1, OUT, TN), lambda b, t: (b, 0, t)),
        out_shape=jax.ShapeDtypeStruct((B, OUT, N1), jnp.float32),
    )(y, stats, gamma.reshape(OUT, 1), beta.reshape(OUT, 1))
    return out


# speedup vs baseline: 1.1998x; 1.1774x over previous
import functools

import jax
import jax.numpy as jnp
from jax.experimental import pallas as pl
from jax.experimental.pallas import tpu as pltpu


def _tree_min(x):
    n = x.shape[1]
    while x.shape[0] > 8:
        g = 8 if x.shape[0] % 64 == 0 else 2
        x = jnp.min(x.reshape(g, x.shape[0] // g, n), axis=0)
    return jnp.min(x, axis=0, keepdims=True)


def _fprop_kernel(xyz1a_ref, xyz2a_ref, a2_ref, b2_ref, f1_ref, f2_ref,
                  wi_ref, wf_ref, y_ref, stats_ref, g2_ref):
    b = pl.program_id(0)
    t = pl.program_id(1)

    @pl.when(jnp.logical_and(b == 0, t == 0))
    def _init_stats():
        stats_ref[...] = jnp.zeros_like(stats_ref)

    @pl.when(t == 0)
    def _compute_g2():
        g2_ref[...] = jnp.dot(wi_ref[...], f2_ref[0],
                              preferred_element_type=jnp.float32)

    cross = jnp.dot(xyz2a_ref[0], xyz1a_ref[0],
                    preferred_element_type=jnp.float32)
    m = cross + b2_ref[0]

    v1 = _tree_min(m)
    m2 = jnp.where(m > v1, m, jnp.float32(3.0e38))
    v2 = _tree_min(m2)
    m3 = jnp.where(m2 > v2, m2, jnp.float32(3.0e38))
    v3 = _tree_min(m3)

    a2 = a2_ref[0]
    w0 = 1.0 / (jnp.sqrt(jnp.maximum(v1 + a2, 1e-12)) + 1e-8)
    w1 = 1.0 / (jnp.sqrt(jnp.maximum(v2 + a2, 1e-12)) + 1e-8)
    w2 = 1.0 / (jnp.sqrt(jnp.maximum(v3 + a2, 1e-12)) + 1e-8)
    ws = 1.0 / (w0 + w1 + w2)
    w0 = w0 * ws
    w1 = w1 * ws
    w2 = w2 * ws

    st = jnp.where(m == v1, w0, 0.0)
    st = jnp.where(m == v2, w1, st)
    st = jnp.where(m == v3, w2, st)

    y = jnp.dot(g2_ref[...], st, preferred_element_type=jnp.float32)
    y = y + jnp.dot(wf_ref[...], f1_ref[0], preferred_element_type=jnp.float32)
    y_ref[0] = y.astype(jnp.bfloat16)
    stats_ref[:, 0:1] += jnp.sum(y, axis=1, keepdims=True)
    stats_ref[:, 1:2] += jnp.sum(y * y, axis=1, keepdims=True)


def _bn_kernel(y_ref, stats_ref, gamma_ref, beta_ref, o_ref, *, count):
    s1 = stats_ref[:, 0:1]
    s2 = stats_ref[:, 1:2]
    mean = s1 * (1.0 / count)
    var = s2 * (1.0 / count) - mean * mean
    a = gamma_ref[...] * jax.lax.rsqrt(var + 1e-5)
    c = beta_ref[...] - a * mean
    o_ref[0] = jnp.maximum(y_ref[0].astype(jnp.float32) * a + c, 0.0)


def kernel(xyz1, xyz2, feats1, feats2, W, gamma, beta):
    B, N1, _ = xyz1.shape
    N2 = xyz2.shape[1]
    C1 = feats1.shape[1]
    C2 = feats2.shape[1]
    OUT = W.shape[0]
    TN = 1024 if N1 % 1024 == 0 else N1
    NT = N1 // TN

    xyz1a = jnp.concatenate(
        [xyz1, jnp.zeros((B, N1, 5), xyz1.dtype)], axis=-1)
    xyz1a = jnp.transpose(xyz1a, (0, 2, 1))
    a2 = jnp.sum(xyz1 * xyz1, axis=-1)[:, None, :]
    b2 = jnp.sum(xyz2 * xyz2, axis=-1, keepdims=True)
    xyz2a = jnp.concatenate(
        [-2.0 * xyz2, jnp.zeros((B, N2, 5), xyz2.dtype)], axis=-1)
    Wi = W[:, :C2]
    Wf = W[:, C2:]

    y, stats = pl.pallas_call(
        _fprop_kernel,
        grid=(B, NT),
        in_specs=[
            pl.BlockSpec((1, 8, TN), lambda b, t: (b, 0, t)),
            pl.BlockSpec((1, N2, 8), lambda b, t: (b, 0, 0)),
            pl.BlockSpec((1, 1, TN), lambda b, t: (b, 0, t)),
            pl.BlockSpec((1, N2, 1), lambda b, t: (b, 0, 0)),
            pl.BlockSpec((1, C1, TN), lambda b, t: (b, 0, t)),
            pl.BlockSpec((1, C2, N2), lambda b, t: (b, 0, 0)),
            pl.BlockSpec((OUT, C2), lambda b, t: (0, 0)),
            pl.BlockSpec((OUT, C1), lambda b, t: (0, 0)),
        ],
        out_specs=[
            pl.BlockSpec((1, OUT, TN), lambda b, t: (b, 0, t)),
            pl.BlockSpec((OUT, 2), lambda b, t: (0, 0)),
        ],
        out_shape=[
            jax.ShapeDtypeStruct((B, OUT, N1), jnp.bfloat16),
            jax.ShapeDtypeStruct((OUT, 2), jnp.float32),
        ],
        scratch_shapes=[pltpu.VMEM((OUT, N2), jnp.float32)],
    )(xyz1a, xyz2a, a2, b2, feats1, feats2, Wi, Wf)

    out = pl.pallas_call(
        functools.partial(_bn_kernel, count=float(B * N1)),
        grid=(B, NT),
        in_specs=[
            pl.BlockSpec((1, OUT, TN), lambda b, t: (b, 0, t)),
            pl.BlockSpec((OUT, 2), lambda b, t: (0, 0)),
            pl.BlockSpec((OUT, 1), lambda b, t: (0, 0)),
            pl.BlockSpec((OUT, 1), lambda b, t: (0, 0)),
        ],
        out_specs=pl.BlockSpec((1, OUT, TN), lambda b, t: (b, 0, t)),
        out_shape=jax.ShapeDtypeStruct((B, OUT, N1), jnp.float32),
    )(y, stats, gamma.reshape(OUT, 1), beta.reshape(OUT, 1))
    return out


# speedup vs baseline: 1.2725x; 1.0605x over previous
import functools

import jax
import jax.numpy as jnp
from jax.experimental import pallas as pl
from jax.experimental.pallas import tpu as pltpu


def _tree_min(x):
    n = x.shape[1]
    while x.shape[0] > 8:
        g = 8 if x.shape[0] % 64 == 0 else 2
        x = jnp.min(x.reshape(g, x.shape[0] // g, n), axis=0)
    return jnp.min(x, axis=0, keepdims=True)


def _fprop_kernel(xyz1a_ref, xyz2a_ref, a2_ref, b2_ref, f1_ref, f2_ref,
                  wi_ref, wf_ref, y_ref, stats_ref, g2_ref):
    b = pl.program_id(0)
    t = pl.program_id(1)

    @pl.when(jnp.logical_and(b == 0, t == 0))
    def _init_stats():
        stats_ref[...] = jnp.zeros_like(stats_ref)

    @pl.when(t == 0)
    def _compute_g2():
        g2_ref[...] = jnp.dot(wi_ref[...], f2_ref[0],
                              preferred_element_type=jnp.float32)

    cross = jnp.dot(xyz2a_ref[0], xyz1a_ref[0],
                    preferred_element_type=jnp.float32)
    m = cross + b2_ref[0]

    v1 = _tree_min(m)
    m2 = jnp.where(m > v1, m, jnp.float32(3.0e38))
    v2 = _tree_min(m2)
    m3 = jnp.where(m2 > v2, m2, jnp.float32(3.0e38))
    v3 = _tree_min(m3)

    a2 = a2_ref[0]
    w0 = 1.0 / (jnp.sqrt(jnp.maximum(v1 + a2, 1e-12)) + 1e-8)
    w1 = 1.0 / (jnp.sqrt(jnp.maximum(v2 + a2, 1e-12)) + 1e-8)
    w2 = 1.0 / (jnp.sqrt(jnp.maximum(v3 + a2, 1e-12)) + 1e-8)
    ws = 1.0 / (w0 + w1 + w2)
    w0 = w0 * ws
    w1 = w1 * ws
    w2 = w2 * ws

    st = jnp.where(m == v1, w0, 0.0)
    st = jnp.where(m == v2, w1, st)
    st = jnp.where(m == v3, w2, st)

    y = jnp.dot(g2_ref[...], st, preferred_element_type=jnp.float32)
    y = y + jnp.dot(wf_ref[...], f1_ref[0], preferred_element_type=jnp.float32)
    y_ref[0] = y.astype(jnp.bfloat16)
    stats_ref[:, 0:1] += jnp.sum(y, axis=1, keepdims=True)
    stats_ref[:, 1:2] += jnp.sum(y * y, axis=1, keepdims=True)


def _bn_kernel(y_ref, stats_ref, gamma_ref, beta_ref, o_ref, *, count):
    s1 = stats_ref[:, 0:1]
    s2 = stats_ref[:, 1:2]
    mean = s1 * (1.0 / count)
    var = s2 * (1.0 / count) - mean * mean
    a = gamma_ref[...] * jax.lax.rsqrt(var + 1e-5)
    c = beta_ref[...] - a * mean
    o_ref[0] = jnp.maximum(y_ref[0].astype(jnp.float32) * a + c, 0.0)


def kernel(xyz1, xyz2, feats1, feats2, W, gamma, beta):
    B, N1, _ = xyz1.shape
    N2 = xyz2.shape[1]
    C1 = feats1.shape[1]
    C2 = feats2.shape[1]
    OUT = W.shape[0]
    TN = 2048 if N1 % 2048 == 0 else N1
    NT = N1 // TN

    xyz1a = jnp.concatenate(
        [xyz1, jnp.zeros((B, N1, 5), xyz1.dtype)], axis=-1)
    xyz1a = jnp.transpose(xyz1a, (0, 2, 1))
    a2 = jnp.sum(xyz1 * xyz1, axis=-1)[:, None, :]
    b2 = jnp.sum(xyz2 * xyz2, axis=-1, keepdims=True)
    xyz2a = jnp.concatenate(
        [-2.0 * xyz2, jnp.zeros((B, N2, 5), xyz2.dtype)], axis=-1)
    Wi = W[:, :C2]
    Wf = W[:, C2:]

    y, stats = pl.pallas_call(
        _fprop_kernel,
        grid=(B, NT),
        in_specs=[
            pl.BlockSpec((1, 8, TN), lambda b, t: (b, 0, t)),
            pl.BlockSpec((1, N2, 8), lambda b, t: (b, 0, 0)),
            pl.BlockSpec((1, 1, TN), lambda b, t: (b, 0, t)),
            pl.BlockSpec((1, N2, 1), lambda b, t: (b, 0, 0)),
            pl.BlockSpec((1, C1, TN), lambda b, t: (b, 0, t)),
            pl.BlockSpec((1, C2, N2), lambda b, t: (b, 0, 0)),
            pl.BlockSpec((OUT, C2), lambda b, t: (0, 0)),
            pl.BlockSpec((OUT, C1), lambda b, t: (0, 0)),
        ],
        out_specs=[
            pl.BlockSpec((1, OUT, TN), lambda b, t: (b, 0, t)),
            pl.BlockSpec((OUT, 2), lambda b, t: (0, 0)),
        ],
        out_shape=[
            jax.ShapeDtypeStruct((B, OUT, N1), jnp.bfloat16),
            jax.ShapeDtypeStruct((OUT, 2), jnp.float32),
        ],
        scratch_shapes=[pltpu.VMEM((OUT, N2), jnp.float32)],
    )(xyz1a, xyz2a, a2, b2, feats1, feats2, Wi, Wf)

    out = pl.pallas_call(
        functools.partial(_bn_kernel, count=float(B * N1)),
        grid=(B, NT),
        in_specs=[
            pl.BlockSpec((1, OUT, TN), lambda b, t: (b, 0, t)),
            pl.BlockSpec((OUT, 2), lambda b, t: (0, 0)),
            pl.BlockSpec((OUT, 1), lambda b, t: (0, 0)),
            pl.BlockSpec((OUT, 1), lambda b, t: (0, 0)),
        ],
        out_specs=pl.BlockSpec((1, OUT, TN), lambda b, t: (b, 0, t)),
        out_shape=jax.ShapeDtypeStruct((B, OUT, N1), jnp.float32),
    )(y, stats, gamma.reshape(OUT, 1), beta.reshape(OUT, 1))
    return out


# speedup vs baseline: 1.3536x; 1.0638x over previous
import functools

import jax
import jax.numpy as jnp
from jax.experimental import pallas as pl
from jax.experimental.pallas import tpu as pltpu


def _tree_min(x):
    n = x.shape[1]
    while x.shape[0] > 8:
        g = 8 if x.shape[0] % 64 == 0 else 2
        x = jnp.min(x.reshape(g, x.shape[0] // g, n), axis=0)
    return jnp.min(x, axis=0, keepdims=True)


def _fprop_kernel(xyz1a_ref, xyz2a_ref, a2_ref, b2_ref, f1_ref, f2_ref,
                  wi_ref, wf_ref, y_ref, stats_ref, g2_ref):
    b = pl.program_id(0)
    t = pl.program_id(1)

    @pl.when(jnp.logical_and(b == 0, t == 0))
    def _init_stats():
        stats_ref[...] = jnp.zeros_like(stats_ref)

    @pl.when(t == 0)
    def _compute_g2():
        g2_ref[...] = jnp.dot(wi_ref[...], f2_ref[0],
                              preferred_element_type=jnp.float32)

    cross = jnp.dot(xyz2a_ref[0], xyz1a_ref[0],
                    preferred_element_type=jnp.float32)
    m = cross + b2_ref[0]

    v1 = _tree_min(m)
    m2 = jnp.where(m > v1, m, jnp.float32(3.0e38))
    v2 = _tree_min(m2)
    m3 = jnp.where(m2 > v2, m2, jnp.float32(3.0e38))
    v3 = _tree_min(m3)

    a2 = a2_ref[0]
    w0 = 1.0 / (jnp.sqrt(jnp.maximum(v1 + a2, 1e-12)) + 1e-8)
    w1 = 1.0 / (jnp.sqrt(jnp.maximum(v2 + a2, 1e-12)) + 1e-8)
    w2 = 1.0 / (jnp.sqrt(jnp.maximum(v3 + a2, 1e-12)) + 1e-8)
    ws = 1.0 / (w0 + w1 + w2)
    w0 = w0 * ws
    w1 = w1 * ws
    w2 = w2 * ws

    st = jnp.where(m == v1, w0, 0.0)
    st = jnp.where(m == v2, w1, st)
    st = jnp.where(m == v3, w2, st)

    y = jnp.dot(g2_ref[...], st, preferred_element_type=jnp.float32)
    y = y + jnp.dot(wf_ref[...], f1_ref[0], preferred_element_type=jnp.float32)
    y_ref[0] = y.astype(jnp.bfloat16)
    stats_ref[:, 0:1] += jnp.sum(y, axis=1, keepdims=True)
    stats_ref[:, 1:2] += jnp.sum(y * y, axis=1, keepdims=True)


def _bn_kernel(y_ref, stats_ref, gamma_ref, beta_ref, o_ref, *, count):
    s1 = stats_ref[:, 0:1]
    s2 = stats_ref[:, 1:2]
    mean = s1 * (1.0 / count)
    var = s2 * (1.0 / count) - mean * mean
    a = gamma_ref[...] * jax.lax.rsqrt(var + 1e-5)
    c = beta_ref[...] - a * mean
    o_ref[0] = jnp.maximum(y_ref[0].astype(jnp.float32) * a + c, 0.0)


def kernel(xyz1, xyz2, feats1, feats2, W, gamma, beta):
    B, N1, _ = xyz1.shape
    N2 = xyz2.shape[1]
    C1 = feats1.shape[1]
    C2 = feats2.shape[1]
    OUT = W.shape[0]
    TN = 4096 if N1 % 4096 == 0 else N1
    NT = N1 // TN

    xyz1a = jnp.concatenate(
        [xyz1, jnp.zeros((B, N1, 5), xyz1.dtype)], axis=-1)
    xyz1a = jnp.transpose(xyz1a, (0, 2, 1))
    a2 = jnp.sum(xyz1 * xyz1, axis=-1)[:, None, :]
    b2 = jnp.sum(xyz2 * xyz2, axis=-1, keepdims=True)
    xyz2a = jnp.concatenate(
        [-2.0 * xyz2, jnp.zeros((B, N2, 5), xyz2.dtype)], axis=-1)
    Wi = W[:, :C2]
    Wf = W[:, C2:]

    y, stats = pl.pallas_call(
        _fprop_kernel,
        grid=(B, NT),
        in_specs=[
            pl.BlockSpec((1, 8, TN), lambda b, t: (b, 0, t)),
            pl.BlockSpec((1, N2, 8), lambda b, t: (b, 0, 0)),
            pl.BlockSpec((1, 1, TN), lambda b, t: (b, 0, t)),
            pl.BlockSpec((1, N2, 1), lambda b, t: (b, 0, 0)),
            pl.BlockSpec((1, C1, TN), lambda b, t: (b, 0, t)),
            pl.BlockSpec((1, C2, N2), lambda b, t: (b, 0, 0)),
            pl.BlockSpec((OUT, C2), lambda b, t: (0, 0)),
            pl.BlockSpec((OUT, C1), lambda b, t: (0, 0)),
        ],
        out_specs=[
            pl.BlockSpec((1, OUT, TN), lambda b, t: (b, 0, t)),
            pl.BlockSpec((OUT, 2), lambda b, t: (0, 0)),
        ],
        out_shape=[
            jax.ShapeDtypeStruct((B, OUT, N1), jnp.bfloat16),
            jax.ShapeDtypeStruct((OUT, 2), jnp.float32),
        ],
        scratch_shapes=[pltpu.VMEM((OUT, N2), jnp.float32)],
    )(xyz1a, xyz2a, a2, b2, feats1, feats2, Wi, Wf)

    out = pl.pallas_call(
        functools.partial(_bn_kernel, count=float(B * N1)),
        grid=(B, NT),
        in_specs=[
            pl.BlockSpec((1, OUT, TN), lambda b, t: (b, 0, t)),
            pl.BlockSpec((OUT, 2), lambda b, t: (0, 0)),
            pl.BlockSpec((OUT, 1), lambda b, t: (0, 0)),
            pl.BlockSpec((OUT, 1), lambda b, t: (0, 0)),
        ],
        out_specs=pl.BlockSpec((1, OUT, TN), lambda b, t: (b, 0, t)),
        out_shape=jax.ShapeDtypeStruct((B, OUT, N1), jnp.float32),
    )(y, stats, gamma.reshape(OUT, 1), beta.reshape(OUT, 1))
    return out
